# Initial kernel scaffold; baseline (speedup 1.0000x reference)
import jax
import jax.numpy as jnp
from jax.experimental import pallas as pl


def kernel(input, table):
    raise NotImplementedError("write your pallas kernel here")


# speedup vs baseline: 7.8613x; 7.8613x over previous
import functools

import jax
import jax.numpy as jnp
from jax import lax
from jax.experimental import pallas as pl
from jax.experimental.pallas import tpu as pltpu
from jax.experimental.pallas import tpu_sc as plsc

B = 4096
E = 128
L = 50
LP = 56
NC = 2
NS = 16
NW = NC * NS
RPW = B // NW
LANES = 16


def _reduce_block(rbuf, outb, r):
    for k in range(E // LANES):
        sl = pl.ds(k * LANES, LANES)
        acc = rbuf[0, sl]
        for j in range(1, LP):
            acc = jnp.maximum(acc, rbuf[j, sl])
        e = jnp.exp(acc * 2.0)
        outb[r, sl] = 1.0 - 2.0 / (e + 1.0)


def _make_sc_kernel():
    mesh = plsc.VectorSubcoreMesh(core_axis_name="c", subcore_axis_name="s")

    @functools.partial(
        pl.kernel,
        out_type=jax.ShapeDtypeStruct((B, E), jnp.float32),
        mesh=mesh,
        scratch_types=[
            pltpu.VMEM((RPW * LP,), jnp.int32),
            pltpu.VMEM((LP, E), jnp.float32),
            pltpu.VMEM((LP, E), jnp.float32),
            pltpu.VMEM((RPW, E), jnp.float32),
            pltpu.SemaphoreType.DMA,
            pltpu.SemaphoreType.DMA,
        ],
    )
    def sc_kernel(idx_hbm, table_hbm, out_hbm, slab, rows0, rows1, outb,
                  sem0, sem1):
        wid = lax.axis_index("s") * NC + lax.axis_index("c")
        base = wid * RPW

        slab_off = pl.multiple_of(base * LP, 8)
        pltpu.sync_copy(idx_hbm.at[pl.ds(slab_off, RPW * LP)], slab)

        def start(c, rbuf, sem):
            off = pl.multiple_of(c * LP, 8)
            idxv = slab.at[pl.ds(off, LP)]
            pltpu.async_copy(table_hbm.at[idxv], rbuf, sem)

        def wait(rbuf, sem):
            pltpu.make_async_copy(
                table_hbm.at[pl.ds(0, LP)], rbuf, sem).wait()

        start(0, rows0, sem0)
        start(1, rows1, sem1)

        def body(i, carry):
            a = 2 * i
            wait(rows0, sem0)
            _reduce_block(rows0, outb, a)
            start(a + 2, rows0, sem0)
            wait(rows1, sem1)
            _reduce_block(rows1, outb, a + 1)
            start(a + 3, rows1, sem1)
            return carry

        lax.fori_loop(0, RPW // 2 - 1, body, 0)

        wait(rows0, sem0)
        _reduce_block(rows0, outb, RPW - 2)
        wait(rows1, sem1)
        _reduce_block(rows1, outb, RPW - 1)

        pltpu.sync_copy(outb, out_hbm.at[pl.ds(base, RPW)])

    return sc_kernel


_sc_kernel = _make_sc_kernel()


@jax.jit
def kernel(input, table):
    inp = input.astype(jnp.int32)
    inp_p = jnp.concatenate([inp, inp[:, : LP - L]], axis=1)
    idx_flat = inp_p.reshape(-1)
    return _sc_kernel(idx_flat, table)
